# Initial kernel scaffold; baseline (speedup 1.0000x reference)
import jax
import jax.numpy as jnp
from jax.experimental import pallas as pl


def kernel(x, table):
    raise NotImplementedError("write your pallas kernel here")


# speedup vs baseline: 3.2263x; 3.2263x over previous
import functools

import jax
import jax.numpy as jnp
from jax import lax
from jax.experimental import pallas as pl
from jax.experimental.pallas import tpu as pltpu
from jax.experimental.pallas import tpu_sc as plsc

_NW = 32
_CHUNK = 512


@functools.partial(jax.jit, static_argnames=("n_rows", "d"))
def _sc_embed(table, idx_flat, n_rows, d):
    b_total = idx_flat.shape[0]
    b_per_w = b_total // _NW
    n_chunks = b_per_w // _CHUNK
    mesh = plsc.VectorSubcoreMesh(core_axis_name="c", subcore_axis_name="s")

    @functools.partial(
        pl.kernel,
        mesh=mesh,
        out_type=jax.ShapeDtypeStruct((b_total, d), jnp.float32),
        scratch_types=[
            pltpu.VMEM((_CHUNK,), jnp.int32),
            pltpu.VMEM((_CHUNK, d), jnp.float32),
            pltpu.SemaphoreType.DMA,
        ],
        compiler_params=pltpu.CompilerParams(use_tc_tiling_on_sc=False),
    )
    def k(table_hbm, idx_hbm, out_hbm, idx_v, rows_v, sem):
        cid = lax.axis_index("c")
        sid = lax.axis_index("s")
        wid = sid * 2 + cid
        base0 = wid * b_per_w

        def body(i, carry):
            base = base0 + i * _CHUNK
            pltpu.sync_copy(idx_hbm.at[pl.ds(base, _CHUNK)], idx_v)
            pltpu.async_copy(table_hbm.at[idx_v], rows_v, sem).wait()
            pltpu.sync_copy(rows_v, out_hbm.at[pl.ds(base, _CHUNK)])
            return carry

        lax.fori_loop(0, n_chunks, body, 0)

    return k(table, idx_flat)


def kernel(x, table):
    s0, s1 = x.shape
    d = table.shape[1]
    idx_flat = x.reshape(-1).astype(jnp.int32)
    out = _sc_embed(table, idx_flat, s0 * s1, d)
    return out.reshape(s0, s1, d)


# speedup vs baseline: 3.2389x; 1.0039x over previous
import functools

import jax
import jax.numpy as jnp
from jax import lax
from jax.experimental import pallas as pl
from jax.experimental.pallas import tpu as pltpu
from jax.experimental.pallas import tpu_sc as plsc

_NW = 32
_CHUNK = 800
_NBUF = 2


@functools.partial(jax.jit, static_argnames=("n_rows", "d"))
def _sc_embed(table, idx_flat, n_rows, d):
    b_total = idx_flat.shape[0]
    b_per_w = b_total // _NW
    n_chunks = b_per_w // _CHUNK
    mesh = plsc.VectorSubcoreMesh(core_axis_name="c", subcore_axis_name="s")

    @functools.partial(
        pl.kernel,
        mesh=mesh,
        out_type=jax.ShapeDtypeStruct((b_total, d), jnp.float32),
        scratch_types=[
            pltpu.VMEM((_CHUNK,), jnp.int32),
            pltpu.VMEM((_CHUNK,), jnp.int32),
            pltpu.VMEM((_CHUNK, d), jnp.float32),
            pltpu.VMEM((_CHUNK, d), jnp.float32),
            pltpu.SemaphoreType.DMA,
            pltpu.SemaphoreType.DMA,
            pltpu.SemaphoreType.DMA,
            pltpu.SemaphoreType.DMA,
            pltpu.SemaphoreType.DMA,
        ],
        compiler_params=pltpu.CompilerParams(use_tc_tiling_on_sc=False),
    )
    def k(table_hbm, idx_hbm, out_hbm, idx_v0, idx_v1, rows_v0, rows_v1,
          sem_i0, sem_i1, sem_g, sem_o0, sem_o1):
        cid = lax.axis_index("c")
        sid = lax.axis_index("s")
        wid = sid * 2 + cid
        base0 = wid * b_per_w
        idx_v = (idx_v0, idx_v1)
        rows_v = (rows_v0, rows_v1)
        sem_i = (sem_i0, sem_i1)
        sem_o = (sem_o0, sem_o1)

        def idx_copy(i, b):
            return pltpu.make_async_copy(
                idx_hbm.at[pl.ds(base0 + i * _CHUNK, _CHUNK)],
                idx_v[b], sem_i[b])

        def gather_copy(b):
            return pltpu.make_async_copy(
                table_hbm.at[idx_v[b]], rows_v[b], sem_g)

        def out_copy(i, b):
            return pltpu.make_async_copy(
                rows_v[b],
                out_hbm.at[pl.ds(base0 + i * _CHUNK, _CHUNK)], sem_o[b])

        idx_copy(0, 0).start()
        idx_copy(1, 1).start()

        @pl.loop(0, n_chunks, step=_NBUF)
        def step(g):
            for b in range(_NBUF):
                i = g + b
                @pl.when(i >= _NBUF)
                def _():
                    out_copy(i - _NBUF, b).wait()

                idx_copy(i, b).wait()
                gather_copy(b).start()
                gather_copy(b).wait()
                out_copy(i, b).start()

                @pl.when(i + _NBUF < n_chunks)
                def _():
                    idx_copy(i + _NBUF, b).start()

        out_copy(n_chunks - 2, (n_chunks - 2) % _NBUF).wait()
        out_copy(n_chunks - 1, (n_chunks - 1) % _NBUF).wait()

    return k(table, idx_flat)


def kernel(x, table):
    s0, s1 = x.shape
    d = table.shape[1]
    idx_flat = x.reshape(-1).astype(jnp.int32)
    out = _sc_embed(table, idx_flat, s0 * s1, d)
    return out.reshape(s0, s1, d)
